# baseline (speedup 1.0000x reference)
import functools

import jax
import jax.numpy as jnp
from jax import lax
from jax.experimental import pallas as pl
from jax.experimental.pallas import tpu as pltpu
from jax.experimental.pallas import tpu_sc as plsc

VOCAB = 100000
HIDDEN = 768
BATCH = 4
SEQ = 2048
EPS = 1e-12

L = 16
NC, NS = 2, 16
NW = NC * NS
PPW = SEQ // NW
NVH = HIDDEN // L

_DNUMS = lax.GatherDimensionNumbers(offset_dims=(), collapsed_slice_dims=(0,),
                                    start_index_map=(0,))


def _xlane_sum(v):
    idx = lax.iota(jnp.int32, L)
    for sh in (1, 2, 4, 8):
        perm = jnp.bitwise_xor(idx, sh)
        v = v + lax.gather(v, perm[:, None], _DNUMS, slice_sizes=(1,),
                           mode=lax.GatherScatterMode.PROMISE_IN_BOUNDS)
    return v


def _rsqrt(x):
    i = lax.bitcast_convert_type(x, jnp.int32)
    i = jnp.int32(0x5F3759DF) - lax.shift_right_arithmetic(i, 1)
    y = lax.bitcast_convert_type(i, jnp.float32)
    for _ in range(3):
        y = y * (1.5 - 0.5 * x * y * y)
    return y


def _sc_body(ids_hbm, tt_hbm, word_hbm, pos_hbm, typ_hbm, scale_hbm, bias_hbm,
             out_hbm, idx_v, tt_v, buf, posb, tbuf, dv, sv, bv):
    wid = lax.axis_index("s") * NC + lax.axis_index("c")
    pbase = wid * PPW

    pltpu.sync_copy(scale_hbm, sv)
    pltpu.sync_copy(bias_hbm, bv)
    pltpu.sync_copy(typ_hbm, tbuf)
    pltpu.sync_copy(pos_hbm.at[pl.ds(pbase, PPW)], posb)

    def prep_j(j, _):
        sl = pl.ds(j * L, L)
        dv[sl] = tbuf[1, sl] - tbuf[0, sl]

        def prep_i(i, _):
            posb[i, sl] = posb[i, sl] + tbuf[0, sl]
            return 0
        lax.fori_loop(0, PPW, prep_i, 0)
        return 0
    lax.fori_loop(0, NVH, prep_j, 0)

    for b in range(BATCH):
        off = b * SEQ + pbase
        pltpu.sync_copy(ids_hbm.at[pl.ds(off, PPW)], idx_v)
        pltpu.sync_copy(tt_hbm.at[pl.ds(off, PPW)], tt_v)
        pltpu.sync_copy(word_hbm.at[idx_v], buf)

        def group_body(g, _):
            tt16 = tt_v[pl.ds(g * L, L)].astype(jnp.float32)
            for l in range(L):
                i = g * L + l
                perm = jnp.full((L, 1), l, jnp.int32)
                ttf = lax.gather(tt16, perm, _DNUMS, slice_sizes=(1,),
                                 mode=lax.GatherScatterMode.PROMISE_IN_BOUNDS)

                def p1(j, acc):
                    sl = pl.ds(j * L, L)
                    v = buf[i, sl] + posb[i, sl] + ttf * dv[sl]
                    buf[i, sl] = v
                    return acc + v
                acc1 = lax.fori_loop(0, NVH, p1, jnp.zeros((L,), jnp.float32))
                mean_v = _xlane_sum(acc1) * (1.0 / HIDDEN)

                def p2(j, acc):
                    d = buf[i, pl.ds(j * L, L)] - mean_v
                    return acc + d * d
                acc2 = lax.fori_loop(0, NVH, p2, jnp.zeros((L,), jnp.float32))
                var_v = _xlane_sum(acc2) * (1.0 / HIDDEN)
                rstd_v = _rsqrt(var_v + EPS)

                def p3(j, _):
                    sl = pl.ds(j * L, L)
                    buf[i, sl] = ((buf[i, sl] - mean_v) * rstd_v * sv[sl]
                                  + bv[sl])
                    return 0
                lax.fori_loop(0, NVH, p3, 0)
            return 0

        lax.fori_loop(0, PPW // L, group_body, 0)
        pltpu.sync_copy(buf, out_hbm.at[pl.ds(off, PPW)])


@jax.jit
def _sc_embed_ln(ids_flat, tt_flat, word_emb, pos_emb, type_emb, ln_scale,
                 ln_bias):
    mesh = plsc.VectorSubcoreMesh(core_axis_name="c", subcore_axis_name="s")
    f = functools.partial(
        pl.kernel,
        out_type=jax.ShapeDtypeStruct((BATCH * SEQ, HIDDEN), jnp.float32),
        mesh=mesh,
        scratch_types=[
            pltpu.VMEM((PPW,), jnp.int32),
            pltpu.VMEM((PPW,), jnp.int32),
            pltpu.VMEM((PPW, HIDDEN), jnp.float32),
            pltpu.VMEM((PPW, HIDDEN), jnp.float32),
            pltpu.VMEM((2, HIDDEN), jnp.float32),
            pltpu.VMEM((HIDDEN,), jnp.float32),
            pltpu.VMEM((HIDDEN,), jnp.float32),
            pltpu.VMEM((HIDDEN,), jnp.float32),
        ],
    )(_sc_body)
    return f(ids_flat, tt_flat, word_emb, pos_emb, type_emb, ln_scale, ln_bias)


def kernel(input_ids, token_type_ids, word_emb, pos_emb, type_emb, ln_scale,
           ln_bias):
    b, s = input_ids.shape
    ids_flat = input_ids.reshape(-1).astype(jnp.int32)
    tt_flat = token_type_ids.reshape(-1).astype(jnp.int32)
    out = _sc_embed_ln(ids_flat, tt_flat, word_emb, pos_emb, type_emb,
                       ln_scale, ln_bias)
    return out.reshape(b, s, HIDDEN)


# speedup vs baseline: 1.6932x; 1.6932x over previous
import functools

import jax
import jax.numpy as jnp
from jax import lax
from jax.experimental import pallas as pl
from jax.experimental.pallas import tpu as pltpu
from jax.experimental.pallas import tpu_sc as plsc

VOCAB = 100000
HIDDEN = 768
BATCH = 4
SEQ = 2048
EPS = 1e-12

L = 16
NC, NS = 2, 16
NW = NC * NS
PPW = SEQ // NW
NVH = HIDDEN // L

_DNUMS = lax.GatherDimensionNumbers(offset_dims=(), collapsed_slice_dims=(0,),
                                    start_index_map=(0,))


def _xlane_sum(v):
    idx = lax.iota(jnp.int32, L)
    for sh in (1, 2, 4, 8):
        perm = jnp.bitwise_xor(idx, sh)
        v = v + lax.gather(v, perm[:, None], _DNUMS, slice_sizes=(1,),
                           mode=lax.GatherScatterMode.PROMISE_IN_BOUNDS)
    return v


def _rsqrt(x):
    i = lax.bitcast_convert_type(x, jnp.int32)
    i = jnp.int32(0x5F3759DF) - lax.shift_right_arithmetic(i, 1)
    y = lax.bitcast_convert_type(i, jnp.float32)
    for _ in range(3):
        y = y * (1.5 - 0.5 * x * y * y)
    return y


def _sc_body(ids_hbm, tt_hbm, word_hbm, pos_hbm, typ_hbm, scale_hbm, bias_hbm,
             out_hbm, idx_v, tt_v, buf, posb, tbuf, dv, sv, bv):
    wid = lax.axis_index("s") * NC + lax.axis_index("c")
    pbase = wid * PPW

    pltpu.sync_copy(scale_hbm, sv)
    pltpu.sync_copy(bias_hbm, bv)
    pltpu.sync_copy(typ_hbm, tbuf)
    pltpu.sync_copy(pos_hbm.at[pl.ds(pbase, PPW)], posb)

    def prep_j(j, _):
        sl = pl.ds(j * L, L)
        dv[sl] = tbuf[1, sl] - tbuf[0, sl]
        return 0
    lax.fori_loop(0, NVH, prep_j, 0)

    def prep_i(i, _):
        def prep_ij(j, _):
            sl = pl.ds(j * L, L)
            posb[i, sl] = posb[i, sl] + tbuf[0, sl]
            return 0
        lax.fori_loop(0, NVH, prep_ij, 0)
        return 0
    lax.fori_loop(0, PPW, prep_i, 0)

    for b in range(BATCH):
        off = b * SEQ + pbase
        pltpu.sync_copy(ids_hbm.at[pl.ds(off, PPW)], idx_v)
        pltpu.sync_copy(tt_hbm.at[pl.ds(off, PPW)], tt_v)
        pltpu.sync_copy(word_hbm.at[idx_v], buf)

        SUB = 4
        zero = jnp.zeros((L,), jnp.float32)

        def group_body(g, _):
            tt16 = tt_v[pl.ds(g * L, L)].astype(jnp.float32)
            for l in range(L):
                i = g * L + l
                perm = jnp.full((L, 1), l, jnp.int32)
                ttf = lax.gather(tt16, perm, _DNUMS, slice_sizes=(1,),
                                 mode=lax.GatherScatterMode.PROMISE_IN_BOUNDS)

                @plsc.parallel_loop(0, NVH, step=1, unroll=SUB,
                                    carry=(zero, zero))
                def stats(j, carry):
                    a, q = carry
                    sl = pl.ds(j * L, L)
                    v = buf[i, sl] + posb[i, sl] + ttf * dv[sl]
                    buf[i, sl] = v
                    return (a + v, q + v * v)
                acc_s, acc_q = stats
                mean_v = _xlane_sum(acc_s) * (1.0 / HIDDEN)
                msq_v = _xlane_sum(acc_q) * (1.0 / HIDDEN)
                var_v = msq_v - mean_v * mean_v
                rstd_v = _rsqrt(var_v + EPS)

                @plsc.parallel_loop(0, NVH, step=1, unroll=SUB)
                def norm(j):
                    sl = pl.ds(j * L, L)
                    buf[i, sl] = ((buf[i, sl] - mean_v) * rstd_v * sv[sl]
                                  + bv[sl])
            return 0

        lax.fori_loop(0, PPW // L, group_body, 0)
        pltpu.sync_copy(buf, out_hbm.at[pl.ds(off, PPW)])


@jax.jit
def _sc_embed_ln(ids_flat, tt_flat, word_emb, pos_emb, type_emb, ln_scale,
                 ln_bias):
    mesh = plsc.VectorSubcoreMesh(core_axis_name="c", subcore_axis_name="s")
    f = functools.partial(
        pl.kernel,
        out_type=jax.ShapeDtypeStruct((BATCH * SEQ, HIDDEN), jnp.float32),
        mesh=mesh,
        scratch_types=[
            pltpu.VMEM((PPW,), jnp.int32),
            pltpu.VMEM((PPW,), jnp.int32),
            pltpu.VMEM((PPW, HIDDEN), jnp.float32),
            pltpu.VMEM((PPW, HIDDEN), jnp.float32),
            pltpu.VMEM((2, HIDDEN), jnp.float32),
            pltpu.VMEM((HIDDEN,), jnp.float32),
            pltpu.VMEM((HIDDEN,), jnp.float32),
            pltpu.VMEM((HIDDEN,), jnp.float32),
        ],
    )(_sc_body)
    return f(ids_flat, tt_flat, word_emb, pos_emb, type_emb, ln_scale, ln_bias)


def kernel(input_ids, token_type_ids, word_emb, pos_emb, type_emb, ln_scale,
           ln_bias):
    b, s = input_ids.shape
    ids_flat = input_ids.reshape(-1).astype(jnp.int32)
    tt_flat = token_type_ids.reshape(-1).astype(jnp.int32)
    out = _sc_embed_ln(ids_flat, tt_flat, word_emb, pos_emb, type_emb,
                       ln_scale, ln_bias)
    return out.reshape(b, s, HIDDEN)


# speedup vs baseline: 2.2978x; 1.3571x over previous
import functools

import jax
import jax.numpy as jnp
from jax import lax
from jax.experimental import pallas as pl
from jax.experimental.pallas import tpu as pltpu
from jax.experimental.pallas import tpu_sc as plsc

VOCAB = 100000
HIDDEN = 768
BATCH = 4
SEQ = 2048
EPS = 1e-12

L = 16
NC, NS = 2, 16
NW = NC * NS
PPW = SEQ // NW
NVH = HIDDEN // L

_DNUMS = lax.GatherDimensionNumbers(offset_dims=(), collapsed_slice_dims=(0,),
                                    start_index_map=(0,))


def _xlane_sum(v):
    idx = lax.iota(jnp.int32, L)
    for sh in (1, 2, 4, 8):
        perm = jnp.bitwise_xor(idx, sh)
        v = v + lax.gather(v, perm[:, None], _DNUMS, slice_sizes=(1,),
                           mode=lax.GatherScatterMode.PROMISE_IN_BOUNDS)
    return v


def _rsqrt(x):
    i = lax.bitcast_convert_type(x, jnp.int32)
    i = jnp.int32(0x5F3759DF) - lax.shift_right_arithmetic(i, 1)
    y = lax.bitcast_convert_type(i, jnp.float32)
    for _ in range(3):
        y = y * (1.5 - 0.5 * x * y * y)
    return y


def _sc_body(ids_hbm, tt_hbm, word_hbm, pos_hbm, typ_hbm, scale_hbm, bias_hbm,
             out_hbm, idx_v, tt_v, buf, posb, tbuf, dv, sv, bv):
    wid = lax.axis_index("s") * NC + lax.axis_index("c")
    pbase = wid * PPW

    pltpu.sync_copy(scale_hbm, sv)
    pltpu.sync_copy(bias_hbm, bv)
    pltpu.sync_copy(typ_hbm, tbuf)
    pltpu.sync_copy(pos_hbm.at[pl.ds(pbase, PPW)], posb)

    def prep_j(j, _):
        sl = pl.ds(j * L, L)
        dv[sl] = tbuf[1, sl] - tbuf[0, sl]
        return 0
    lax.fori_loop(0, NVH, prep_j, 0)

    def prep_i(i, _):
        def prep_ij(j, _):
            sl = pl.ds(j * L, L)
            posb[i, sl] = posb[i, sl] + tbuf[0, sl]
            return 0
        lax.fori_loop(0, NVH, prep_ij, 0)
        return 0
    lax.fori_loop(0, PPW, prep_i, 0)

    for b in range(BATCH):
        off = b * SEQ + pbase
        pltpu.sync_copy(ids_hbm.at[pl.ds(off, PPW)], idx_v)
        pltpu.sync_copy(tt_hbm.at[pl.ds(off, PPW)], tt_v)
        pltpu.sync_copy(word_hbm.at[idx_v], buf)

        SUB = 4
        zero = jnp.zeros((L,), jnp.float32)

        def group_body(g, _):
            tt16 = tt_v[pl.ds(g * L, L)].astype(jnp.float32)
            for l in range(0, L, 2):
                i0 = g * L + l
                i1 = i0 + 1
                perm0 = jnp.full((L, 1), l, jnp.int32)
                perm1 = jnp.full((L, 1), l + 1, jnp.int32)
                ttf0 = lax.gather(tt16, perm0, _DNUMS, slice_sizes=(1,),
                                  mode=lax.GatherScatterMode.PROMISE_IN_BOUNDS)
                ttf1 = lax.gather(tt16, perm1, _DNUMS, slice_sizes=(1,),
                                  mode=lax.GatherScatterMode.PROMISE_IN_BOUNDS)

                @plsc.parallel_loop(0, NVH, step=1, unroll=SUB,
                                    carry=(zero, zero, zero, zero))
                def stats(j, carry):
                    a0, q0, a1, q1 = carry
                    sl = pl.ds(j * L, L)
                    d = dv[sl]
                    v0 = buf[i0, sl] + posb[i0, sl] + ttf0 * d
                    v1 = buf[i1, sl] + posb[i1, sl] + ttf1 * d
                    buf[i0, sl] = v0
                    buf[i1, sl] = v1
                    return (a0 + v0, q0 + v0 * v0, a1 + v1, q1 + v1 * v1)
                acc_s0, acc_q0, acc_s1, acc_q1 = stats

                mean0 = _xlane_sum(acc_s0) * (1.0 / HIDDEN)
                mean1 = _xlane_sum(acc_s1) * (1.0 / HIDDEN)
                msq0 = _xlane_sum(acc_q0) * (1.0 / HIDDEN)
                msq1 = _xlane_sum(acc_q1) * (1.0 / HIDDEN)
                rstd0 = _rsqrt(msq0 - mean0 * mean0 + EPS)
                rstd1 = _rsqrt(msq1 - mean1 * mean1 + EPS)

                @plsc.parallel_loop(0, NVH, step=1, unroll=SUB)
                def norm(j):
                    sl = pl.ds(j * L, L)
                    s = sv[sl]
                    bb = bv[sl]
                    buf[i0, sl] = (buf[i0, sl] - mean0) * rstd0 * s + bb
                    buf[i1, sl] = (buf[i1, sl] - mean1) * rstd1 * s + bb
            return 0

        lax.fori_loop(0, PPW // L, group_body, 0)
        pltpu.sync_copy(buf, out_hbm.at[pl.ds(off, PPW)])


@jax.jit
def _sc_embed_ln(ids_flat, tt_flat, word_emb, pos_emb, type_emb, ln_scale,
                 ln_bias):
    mesh = plsc.VectorSubcoreMesh(core_axis_name="c", subcore_axis_name="s")
    f = functools.partial(
        pl.kernel,
        out_type=jax.ShapeDtypeStruct((BATCH * SEQ, HIDDEN), jnp.float32),
        mesh=mesh,
        scratch_types=[
            pltpu.VMEM((PPW,), jnp.int32),
            pltpu.VMEM((PPW,), jnp.int32),
            pltpu.VMEM((PPW, HIDDEN), jnp.float32),
            pltpu.VMEM((PPW, HIDDEN), jnp.float32),
            pltpu.VMEM((2, HIDDEN), jnp.float32),
            pltpu.VMEM((HIDDEN,), jnp.float32),
            pltpu.VMEM((HIDDEN,), jnp.float32),
            pltpu.VMEM((HIDDEN,), jnp.float32),
        ],
    )(_sc_body)
    return f(ids_flat, tt_flat, word_emb, pos_emb, type_emb, ln_scale, ln_bias)


def kernel(input_ids, token_type_ids, word_emb, pos_emb, type_emb, ln_scale,
           ln_bias):
    b, s = input_ids.shape
    ids_flat = input_ids.reshape(-1).astype(jnp.int32)
    tt_flat = token_type_ids.reshape(-1).astype(jnp.int32)
    out = _sc_embed_ln(ids_flat, tt_flat, word_emb, pos_emb, type_emb,
                       ln_scale, ln_bias)
    return out.reshape(b, s, HIDDEN)


# speedup vs baseline: 2.4231x; 1.0545x over previous
import functools

import jax
import jax.numpy as jnp
from jax import lax
from jax.experimental import pallas as pl
from jax.experimental.pallas import tpu as pltpu
from jax.experimental.pallas import tpu_sc as plsc

VOCAB = 100000
HIDDEN = 768
BATCH = 4
SEQ = 2048
EPS = 1e-12

L = 16
NC, NS = 2, 16
NW = NC * NS
PPW = SEQ // NW
TPW = BATCH * PPW
CH = 32
NCHUNK = TPW // CH
NVH = HIDDEN // L
SUB = 4

_DNUMS = lax.GatherDimensionNumbers(offset_dims=(), collapsed_slice_dims=(0,),
                                    start_index_map=(0,))


def _xlane_sum(v):
    idx = lax.iota(jnp.int32, L)
    for sh in (1, 2, 4, 8):
        perm = jnp.bitwise_xor(idx, sh)
        v = v + lax.gather(v, perm[:, None], _DNUMS, slice_sizes=(1,),
                           mode=lax.GatherScatterMode.PROMISE_IN_BOUNDS)
    return v


def _rsqrt(x):
    i = lax.bitcast_convert_type(x, jnp.int32)
    i = jnp.int32(0x5F3759DF) - lax.shift_right_arithmetic(i, 1)
    y = lax.bitcast_convert_type(i, jnp.float32)
    for _ in range(3):
        y = y * (1.5 - 0.5 * x * y * y)
    return y


def _sc_body(ids_hbm, tt_hbm, word_hbm, pos_hbm, typ_hbm, scale_hbm, bias_hbm,
             out_hbm, idx_all, tt_all, buf0, buf1, posb, tbuf, dv, sv, bv,
             gs0, gs1, ws0, ws1):
    wid = lax.axis_index("s") * NC + lax.axis_index("c")
    pbase = wid * PPW
    bufs = (buf0, buf1)
    gsems = (gs0, gs1)
    wsems = (ws0, ws1)

    for b in range(BATCH):
        pltpu.sync_copy(ids_hbm.at[pl.ds(b * SEQ + pbase, PPW)],
                        idx_all.at[pl.ds(b * PPW, PPW)])
        pltpu.sync_copy(tt_hbm.at[pl.ds(b * SEQ + pbase, PPW)],
                        tt_all.at[pl.ds(b * PPW, PPW)])
    pltpu.sync_copy(scale_hbm, sv)
    pltpu.sync_copy(bias_hbm, bv)
    pltpu.sync_copy(typ_hbm, tbuf)
    pltpu.sync_copy(pos_hbm.at[pl.ds(pbase, PPW)], posb)

    gh = {}
    wh = {}
    gh[0] = pltpu.async_copy(word_hbm.at[idx_all.at[pl.ds(0, CH)]], buf0, gs0)
    gh[1] = pltpu.async_copy(word_hbm.at[idx_all.at[pl.ds(CH, CH)]], buf1, gs1)

    def prep_j(j, _):
        sl = pl.ds(j * L, L)
        dv[sl] = tbuf[1, sl] - tbuf[0, sl]
        return 0
    lax.fori_loop(0, NVH, prep_j, 0)

    def prep_i(i, _):
        def prep_ij(j, _):
            sl = pl.ds(j * L, L)
            posb[i, sl] = posb[i, sl] + tbuf[0, sl]
            return 0
        lax.fori_loop(0, NVH, prep_ij, 0)
        return 0
    lax.fori_loop(0, PPW, prep_i, 0)

    zero = jnp.zeros((L,), jnp.float32)

    for c in range(NCHUNK):
        b, h = divmod(c, 2)
        buf = bufs[c % 2]
        gh[c].wait()
        if c + 1 < NCHUNK:
            if c - 1 >= 0:
                wh[c - 1].wait()
            nxt = c + 1
            gh[nxt] = pltpu.async_copy(
                word_hbm.at[idx_all.at[pl.ds(nxt * CH, CH)]],
                bufs[nxt % 2], gsems[nxt % 2])

        def group_body(g, _, _buf=buf, _c=c, _h=h):
            tt16 = tt_all[pl.ds(_c * CH + g * L, L)].astype(jnp.float32)
            for l in range(0, L, 2):
                i0 = g * L + l
                i1 = i0 + 1
                p0r = _h * CH + i0
                p1r = p0r + 1
                perm0 = jnp.full((L, 1), l, jnp.int32)
                perm1 = jnp.full((L, 1), l + 1, jnp.int32)
                ttf0 = lax.gather(tt16, perm0, _DNUMS, slice_sizes=(1,),
                                  mode=lax.GatherScatterMode.PROMISE_IN_BOUNDS)
                ttf1 = lax.gather(tt16, perm1, _DNUMS, slice_sizes=(1,),
                                  mode=lax.GatherScatterMode.PROMISE_IN_BOUNDS)

                @plsc.parallel_loop(0, NVH, step=1, unroll=SUB,
                                    carry=(zero, zero, zero, zero))
                def stats(j, carry):
                    a0, q0, a1, q1 = carry
                    sl = pl.ds(j * L, L)
                    d = dv[sl]
                    v0 = _buf[i0, sl] + posb[p0r, sl] + ttf0 * d
                    v1 = _buf[i1, sl] + posb[p1r, sl] + ttf1 * d
                    _buf[i0, sl] = v0
                    _buf[i1, sl] = v1
                    return (a0 + v0, q0 + v0 * v0, a1 + v1, q1 + v1 * v1)
                acc_s0, acc_q0, acc_s1, acc_q1 = stats

                mean0 = _xlane_sum(acc_s0) * (1.0 / HIDDEN)
                mean1 = _xlane_sum(acc_s1) * (1.0 / HIDDEN)
                msq0 = _xlane_sum(acc_q0) * (1.0 / HIDDEN)
                msq1 = _xlane_sum(acc_q1) * (1.0 / HIDDEN)
                rstd0 = _rsqrt(msq0 - mean0 * mean0 + EPS)
                rstd1 = _rsqrt(msq1 - mean1 * mean1 + EPS)

                @plsc.parallel_loop(0, NVH, step=1, unroll=SUB)
                def norm(j):
                    sl = pl.ds(j * L, L)
                    s = sv[sl]
                    bb = bv[sl]
                    _buf[i0, sl] = (_buf[i0, sl] - mean0) * rstd0 * s + bb
                    _buf[i1, sl] = (_buf[i1, sl] - mean1) * rstd1 * s + bb
            return 0

        lax.fori_loop(0, CH // L, group_body, 0)
        off = b * SEQ + pbase + h * CH
        wh[c] = pltpu.async_copy(buf, out_hbm.at[pl.ds(off, CH)],
                                 wsems[c % 2])

    wh[NCHUNK - 2].wait()
    wh[NCHUNK - 1].wait()


@jax.jit
def _sc_embed_ln(ids_flat, tt_flat, word_emb, pos_emb, type_emb, ln_scale,
                 ln_bias):
    mesh = plsc.VectorSubcoreMesh(core_axis_name="c", subcore_axis_name="s")
    f = functools.partial(
        pl.kernel,
        out_type=jax.ShapeDtypeStruct((BATCH * SEQ, HIDDEN), jnp.float32),
        mesh=mesh,
        scratch_types=[
            pltpu.VMEM((TPW,), jnp.int32),
            pltpu.VMEM((TPW,), jnp.int32),
            pltpu.VMEM((CH, HIDDEN), jnp.float32),
            pltpu.VMEM((CH, HIDDEN), jnp.float32),
            pltpu.VMEM((PPW, HIDDEN), jnp.float32),
            pltpu.VMEM((2, HIDDEN), jnp.float32),
            pltpu.VMEM((HIDDEN,), jnp.float32),
            pltpu.VMEM((HIDDEN,), jnp.float32),
            pltpu.VMEM((HIDDEN,), jnp.float32),
            pltpu.SemaphoreType.DMA,
            pltpu.SemaphoreType.DMA,
            pltpu.SemaphoreType.DMA,
            pltpu.SemaphoreType.DMA,
        ],
    )(_sc_body)
    return f(ids_flat, tt_flat, word_emb, pos_emb, type_emb, ln_scale, ln_bias)


def kernel(input_ids, token_type_ids, word_emb, pos_emb, type_emb, ln_scale,
           ln_bias):
    b, s = input_ids.shape
    ids_flat = input_ids.reshape(-1).astype(jnp.int32)
    tt_flat = token_type_ids.reshape(-1).astype(jnp.int32)
    out = _sc_embed_ln(ids_flat, tt_flat, word_emb, pos_emb, type_emb,
                       ln_scale, ln_bias)
    return out.reshape(b, s, HIDDEN)
